# baseline (speedup 1.0000x reference)
import functools

import jax
import jax.numpy as jnp
from jax import lax
from jax.experimental import pallas as pl
from jax.experimental.pallas import tpu as pltpu
from jax.experimental.pallas import tpu_sc as plsc

_N_NODES = 10000
_N_EDGES = 160000
_N_ANGLES = 320000
_DIM = 128

_NC = 2
_NS = 16
_NW = _NC * _NS
_C = 128
_PAD_PER_W = 64

_mesh = plsc.VectorSubcoreMesh(
    core_axis_name="c", subcore_axis_name="s", num_cores=_NC, num_subcores=_NS)


def _edge_stage_sc(db, eh, ce, src2, dst2, n_edges):
    e_pad = n_edges + _NW * _PAD_PER_W
    ncch = n_edges // _C

    @functools.partial(
        pl.kernel,
        out_type=(
            jax.ShapeDtypeStruct((n_edges, _DIM), jnp.float32),
            jax.ShapeDtypeStruct((e_pad, 2 * _DIM), jnp.float32),
        ),
        mesh=_mesh,
        scratch_types=[
            pltpu.VMEM((_C,), jnp.int32),
            pltpu.VMEM((_C,), jnp.int32),
            pltpu.VMEM((_C, 2 * _DIM), jnp.float32),
            pltpu.VMEM((_C, _DIM), jnp.float32),
            pltpu.VMEM((_C, _DIM), jnp.float32),
            pltpu.VMEM((_C, _DIM), jnp.float32),
            pltpu.VMEM((_C, 2 * _DIM), jnp.float32),
            pltpu.SemaphoreType.DMA,
            pltpu.SemaphoreType.DMA,
        ],
    )
    def k(db_hbm, eh_hbm, ce_hbm, src_hbm, dst_hbm, enew_hbm, contrib_hbm,
          src_v, dst_v, db_v, eh_v, ce_v, enew_v, con_v, sem0, sem1):
        wid = lax.axis_index("s") * _NC + lax.axis_index("c")

        zero16 = jnp.zeros((16,), jnp.float32)
        def _zrow(r, _):
            for v in range(2 * _DIM // 16):
                con_v[r, pl.ds(v * 16, 16)] = zero16
            return 0
        lax.fori_loop(0, _PAD_PER_W, _zrow, 0)
        pltpu.sync_copy(con_v.at[pl.ds(0, _PAD_PER_W)],
                        contrib_hbm.at[pl.ds(n_edges + wid * _PAD_PER_W, _PAD_PER_W)])

        nch_w = (ncch - wid + _NW - 1) // _NW

        def _chunk(i, _):
            ch = wid + i * _NW
            off = ch * _C
            pltpu.sync_copy(src_hbm.at[ch], src_v)
            pltpu.sync_copy(dst_hbm.at[ch], dst_v)
            pltpu.sync_copy(ce_hbm.at[pl.ds(off, _C)], ce_v)
            cp0 = pltpu.async_copy(db_hbm.at[src_v], db_v, sem0)
            cp1 = pltpu.async_copy(eh_hbm.at[dst_v], eh_v, sem1)
            cp0.wait()
            cp1.wait()

            def _row(r, _):
                for v in range(_DIM // 16):
                    sl = pl.ds(v * 16, 16)
                    sl2 = pl.ds(_DIM + v * 16, 16)
                    eji = db_v[r, sl] + eh_v[r, sl] + ce_v[r, sl]
                    sig = 1.0 / (1.0 + jnp.exp(-eji))
                    enew_v[r, sl] = jnp.maximum(eji, 0.0)
                    con_v[r, sl] = sig * db_v[r, sl2]
                    con_v[r, sl2] = sig
                return 0
            lax.fori_loop(0, _C, _row, 0)

            pltpu.sync_copy(enew_v, enew_hbm.at[pl.ds(off, _C)])
            pltpu.sync_copy(con_v, contrib_hbm.at[pl.ds(off, _C)])
            return 0

        lax.fori_loop(0, nch_w, _chunk, 0)

    return k(db, eh, ce, src2, dst2)


def _node_stage_body(ah_ref, num_ref, den_ref, h_ref):
    h_ref[...] = jax.nn.relu(ah_ref[...] + num_ref[...] / (den_ref[...] + 1e-6))


def _node_stage(ah, num, den, block=2000):
    n = ah.shape[0]
    grid = (n // block,)
    spec = pl.BlockSpec((block, _DIM), lambda i: (i, 0))
    return pl.pallas_call(
        _node_stage_body,
        grid=grid,
        in_specs=[spec, spec, spec],
        out_specs=spec,
        out_shape=jax.ShapeDtypeStruct((n, _DIM), jnp.float32),
    )(ah, num, den)


def _gated_layer(h, e, edge_index, p, n_nodes, n_edges):
    src = edge_index[0]
    dst = edge_index[1]
    Ah = h @ p['A'][0] + p['A'][1]
    Bh = h @ p['B'][0] + p['B'][1]
    Dh = h @ p['D'][0] + p['D'][1]
    Eh = h @ p['E'][0] + p['E'][1]
    Ce = e @ p['C'][0] + p['C'][1]
    db = jnp.concatenate([Dh, Bh], axis=1)
    src2 = src.reshape(n_edges // _C, _C)
    dst2 = dst.reshape(n_edges // _C, _C)
    e_new, contrib = _edge_stage_sc(db, Eh, Ce, src2, dst2, n_edges)
    acc = jax.ops.segment_sum(contrib[:n_edges], dst, num_segments=n_nodes)
    h_new = _node_stage(Ah, acc[:, :_DIM], acc[:, _DIM:])
    return h_new, e_new


def kernel(node_feats, edge_feats, angle_feats, graph_edge_index, line_graph_edge_index, params):
    h, e = _gated_layer(node_feats, edge_feats, graph_edge_index,
                        params['node_update'], _N_NODES, _N_EDGES)
    e, a = _gated_layer(e, angle_feats, line_graph_edge_index,
                        params['edge_update'], _N_EDGES, _N_ANGLES)
    return (h, e, a)


# speedup vs baseline: 1.8356x; 1.8356x over previous
import functools

import jax
import jax.numpy as jnp
from jax import lax
from jax.experimental import pallas as pl
from jax.experimental.pallas import tpu as pltpu
from jax.experimental.pallas import tpu_sc as plsc

_N_NODES = 10000
_N_EDGES = 160000
_N_ANGLES = 320000
_DIM = 128

_NC = 2
_NS = 16
_NW = _NC * _NS
_C = 128

_mesh = plsc.VectorSubcoreMesh(
    core_axis_name="c", subcore_axis_name="s", num_cores=_NC, num_subcores=_NS)


def _gather_stage_sc(db, eh, src2, dst2, n_edges):
    nch = n_edges // _C

    @functools.partial(
        pl.kernel,
        out_type=(
            jax.ShapeDtypeStruct((n_edges, 2 * _DIM), jnp.float32),
            jax.ShapeDtypeStruct((n_edges, _DIM), jnp.float32),
        ),
        mesh=_mesh,
        scratch_types=[
            pltpu.VMEM((_C,), jnp.int32), pltpu.VMEM((_C,), jnp.int32),
            pltpu.VMEM((_C,), jnp.int32), pltpu.VMEM((_C,), jnp.int32),
            pltpu.VMEM((_C, 2 * _DIM), jnp.float32),
            pltpu.VMEM((_C, 2 * _DIM), jnp.float32),
            pltpu.VMEM((_C, _DIM), jnp.float32),
            pltpu.VMEM((_C, _DIM), jnp.float32),
            pltpu.SemaphoreType.DMA, pltpu.SemaphoreType.DMA,
            pltpu.SemaphoreType.DMA, pltpu.SemaphoreType.DMA,
            pltpu.SemaphoreType.DMA, pltpu.SemaphoreType.DMA,
        ],
    )
    def k(db_hbm, eh_hbm, src_hbm, dst_hbm, gsrc_hbm, gdst_hbm,
          src0, src1, dst0, dst1, gs0, gs1, gd0, gd1,
          si0, si1, sg0, sg1, sw0, sw1):
        wid = lax.axis_index("s") * _NC + lax.axis_index("c")
        nch_w = (nch - wid + _NW - 1) // _NW
        srcs, dsts = (src0, src1), (dst0, dst1)
        gss, gds = (gs0, gs1), (gd0, gd1)
        sis, sgs, sws = (si0, si1), (sg0, sg1), (sw0, sw1)

        def ch_of(i):
            return wid + i * _NW

        def issue_idx(i, b):
            ch = ch_of(i)
            pltpu.async_copy(src_hbm.at[ch], srcs[b], sis[b])
            pltpu.async_copy(dst_hbm.at[ch], dsts[b], sis[b])

        def wait_idx(b):
            pltpu.make_async_copy(src_hbm.at[0], srcs[b], sis[b]).wait()
            pltpu.make_async_copy(dst_hbm.at[0], dsts[b], sis[b]).wait()

        def issue_gather(b):
            pltpu.async_copy(db_hbm.at[srcs[b]], gss[b], sgs[b])
            pltpu.async_copy(eh_hbm.at[dsts[b]], gds[b], sgs[b])

        def wait_gather(b):
            pltpu.make_async_copy(db_hbm.at[srcs[b]], gss[b], sgs[b]).wait()
            pltpu.make_async_copy(eh_hbm.at[dsts[b]], gds[b], sgs[b]).wait()

        def issue_write(i, b):
            off = ch_of(i) * _C
            pltpu.async_copy(gss[b], gsrc_hbm.at[pl.ds(off, _C)], sws[b])
            pltpu.async_copy(gds[b], gdst_hbm.at[pl.ds(off, _C)], sws[b])

        def wait_write(b):
            pltpu.make_async_copy(gss[b], gsrc_hbm.at[pl.ds(0, _C)], sws[b]).wait()
            pltpu.make_async_copy(gds[b], gdst_hbm.at[pl.ds(0, _C)], sws[b]).wait()

        @pl.when(nch_w > 0)
        def _():
            issue_idx(0, 0)
            wait_idx(0)
            issue_gather(0)

        @pl.when(nch_w > 1)
        def _():
            issue_idx(1, 1)

        def step(i, b, nvalid):
            @pl.when(i + 1 < nvalid)
            def _():
                wait_idx(1 - b)
                @pl.when(i >= 1)
                def _():
                    wait_write(1 - b)
                issue_gather(1 - b)
            @pl.when(i < nvalid)
            def _():
                wait_gather(b)
                issue_write(i, b)
            @pl.when(i + 2 < nvalid)
            def _():
                issue_idx(i + 2, b)

        def pair(i2, _):
            step(2 * i2, 0, nch_w)
            step(2 * i2 + 1, 1, nch_w)
            return 0

        lax.fori_loop(0, (nch_w + 1) // 2, pair, 0)

        @pl.when(nch_w > 1)
        def _():
            wait_write(1)

        @pl.when(nch_w > 0)
        def _():
            wait_write(0)

    return k(db, eh, src2, dst2)


def _mm_body(x_ref, w_ref, b_ref, o_ref):
    o_ref[...] = jnp.dot(x_ref[...], w_ref[...],
                         preferred_element_type=jnp.float32) + b_ref[...]


def _mm(x, w, b, block):
    m, kdim = x.shape
    kout = w.shape[1]
    return pl.pallas_call(
        _mm_body,
        grid=(m // block,),
        in_specs=[pl.BlockSpec((block, kdim), lambda i: (i, 0)),
                  pl.BlockSpec((kdim, kout), lambda i: (0, 0)),
                  pl.BlockSpec((1, kout), lambda i: (0, 0))],
        out_specs=pl.BlockSpec((block, kout), lambda i: (i, 0)),
        out_shape=jax.ShapeDtypeStruct((m, kout), jnp.float32),
    )(x, w, b.reshape(1, kout))


def _edge_stage_body(gs_ref, gd_ref, ce_ref, enew_ref, con_ref):
    d = gs_ref[:, :_DIM]
    bb = gs_ref[:, _DIM:]
    e_ji = d + gd_ref[...] + ce_ref[...]
    sig = jax.nn.sigmoid(e_ji)
    enew_ref[...] = jax.nn.relu(e_ji)
    con_ref[:, :_DIM] = sig * bb
    con_ref[:, _DIM:] = sig


def _edge_stage_tc(gsrc, gdst, ce, block=2000):
    n = gsrc.shape[0]
    return pl.pallas_call(
        _edge_stage_body,
        grid=(n // block,),
        in_specs=[pl.BlockSpec((block, 2 * _DIM), lambda i: (i, 0)),
                  pl.BlockSpec((block, _DIM), lambda i: (i, 0)),
                  pl.BlockSpec((block, _DIM), lambda i: (i, 0))],
        out_specs=[pl.BlockSpec((block, _DIM), lambda i: (i, 0)),
                   pl.BlockSpec((block, 2 * _DIM), lambda i: (i, 0))],
        out_shape=[jax.ShapeDtypeStruct((n, _DIM), jnp.float32),
                   jax.ShapeDtypeStruct((n, 2 * _DIM), jnp.float32)],
    )(gsrc, gdst, ce)


def _node_stage_body(ah_ref, acc_ref, h_ref):
    num = acc_ref[:, :_DIM]
    den = acc_ref[:, _DIM:]
    h_ref[...] = jax.nn.relu(ah_ref[...] + num / (den + 1e-6))


def _node_stage(ah, acc, block=2000):
    n = ah.shape[0]
    return pl.pallas_call(
        _node_stage_body,
        grid=(n // block,),
        in_specs=[pl.BlockSpec((block, _DIM), lambda i: (i, 0)),
                  pl.BlockSpec((block, 2 * _DIM), lambda i: (i, 0))],
        out_specs=pl.BlockSpec((block, _DIM), lambda i: (i, 0)),
        out_shape=jax.ShapeDtypeStruct((n, _DIM), jnp.float32),
    )(ah, acc)


def _gated_layer(h, e, edge_index, p, n_nodes, n_edges, node_block):
    src = edge_index[0]
    dst = edge_index[1]
    wdb = jnp.concatenate([p['D'][0], p['B'][0]], axis=1)
    bdb = jnp.concatenate([p['D'][1], p['B'][1]], axis=0)
    db = _mm(h, wdb, bdb, block=min(2000, n_nodes // 5))
    Eh = _mm(h, p['E'][0], p['E'][1], block=min(2000, n_nodes // 5))
    Ah = _mm(h, p['A'][0], p['A'][1], block=min(2000, n_nodes // 5))
    Ce = _mm(e, p['C'][0], p['C'][1], block=2000)
    src2 = src.reshape(n_edges // _C, _C)
    dst2 = dst.reshape(n_edges // _C, _C)
    gsrc, gdst = _gather_stage_sc(db, Eh, src2, dst2, n_edges)
    e_new, contrib = _edge_stage_tc(gsrc, gdst, Ce)
    acc = jax.ops.segment_sum(contrib, dst, num_segments=n_nodes)
    h_new = _node_stage(Ah, acc, block=node_block)
    return h_new, e_new


def kernel(node_feats, edge_feats, angle_feats, graph_edge_index, line_graph_edge_index, params):
    h, e = _gated_layer(node_feats, edge_feats, graph_edge_index,
                        params['node_update'], _N_NODES, _N_EDGES, node_block=2000)
    e, a = _gated_layer(e, angle_feats, line_graph_edge_index,
                        params['edge_update'], _N_EDGES, _N_ANGLES, node_block=2000)
    return (h, e, a)


# speedup vs baseline: 1.8371x; 1.0008x over previous
import functools

import jax
import jax.numpy as jnp
from jax import lax
from jax.experimental import pallas as pl
from jax.experimental.pallas import tpu as pltpu
from jax.experimental.pallas import tpu_sc as plsc

_N_NODES = 10000
_N_EDGES = 160000
_N_ANGLES = 320000
_DIM = 128

_NC = 2
_NS = 16
_NW = _NC * _NS
_C = 128

_mesh = plsc.VectorSubcoreMesh(
    core_axis_name="c", subcore_axis_name="s", num_cores=_NC, num_subcores=_NS)


def _gather_stage_sc(db, eh, src2, dst2, n_edges):
    nch = n_edges // _C

    @functools.partial(
        pl.kernel,
        out_type=(
            jax.ShapeDtypeStruct((n_edges, 2 * _DIM), jnp.float32),
            jax.ShapeDtypeStruct((n_edges, _DIM), jnp.float32),
        ),
        mesh=_mesh,
        scratch_types=[
            pltpu.VMEM((_C,), jnp.int32), pltpu.VMEM((_C,), jnp.int32),
            pltpu.VMEM((_C,), jnp.int32), pltpu.VMEM((_C,), jnp.int32),
            pltpu.VMEM((_C, 2 * _DIM), jnp.float32),
            pltpu.VMEM((_C, 2 * _DIM), jnp.float32),
            pltpu.VMEM((_C, _DIM), jnp.float32),
            pltpu.VMEM((_C, _DIM), jnp.float32),
            pltpu.SemaphoreType.DMA, pltpu.SemaphoreType.DMA,
            pltpu.SemaphoreType.DMA, pltpu.SemaphoreType.DMA,
            pltpu.SemaphoreType.DMA, pltpu.SemaphoreType.DMA,
        ],
    )
    def k(db_hbm, eh_hbm, src_hbm, dst_hbm, gsrc_hbm, gdst_hbm,
          src0, src1, dst0, dst1, gs0, gs1, gd0, gd1,
          si0, si1, sg0, sg1, sw0, sw1):
        wid = lax.axis_index("s") * _NC + lax.axis_index("c")
        nch_w = (nch - wid + _NW - 1) // _NW
        srcs, dsts = (src0, src1), (dst0, dst1)
        gss, gds = (gs0, gs1), (gd0, gd1)
        sis, sgs, sws = (si0, si1), (sg0, sg1), (sw0, sw1)

        def ch_of(i):
            return wid + i * _NW

        def issue_idx(i, b):
            ch = ch_of(i)
            pltpu.async_copy(src_hbm.at[ch], srcs[b], sis[b])
            pltpu.async_copy(dst_hbm.at[ch], dsts[b], sis[b])

        def wait_idx(b):
            pltpu.make_async_copy(src_hbm.at[0], srcs[b], sis[b]).wait()
            pltpu.make_async_copy(dst_hbm.at[0], dsts[b], sis[b]).wait()

        def issue_gather(b):
            pltpu.async_copy(db_hbm.at[srcs[b]], gss[b], sgs[b])
            pltpu.async_copy(eh_hbm.at[dsts[b]], gds[b], sgs[b])

        def wait_gather(b):
            pltpu.make_async_copy(db_hbm.at[srcs[b]], gss[b], sgs[b]).wait()
            pltpu.make_async_copy(eh_hbm.at[dsts[b]], gds[b], sgs[b]).wait()

        def issue_write(i, b):
            off = ch_of(i) * _C
            pltpu.async_copy(gss[b], gsrc_hbm.at[pl.ds(off, _C)], sws[b])
            pltpu.async_copy(gds[b], gdst_hbm.at[pl.ds(off, _C)], sws[b])

        def wait_write(b):
            pltpu.make_async_copy(gss[b], gsrc_hbm.at[pl.ds(0, _C)], sws[b]).wait()
            pltpu.make_async_copy(gds[b], gdst_hbm.at[pl.ds(0, _C)], sws[b]).wait()

        @pl.when(nch_w > 0)
        def _():
            issue_idx(0, 0)
            wait_idx(0)
            issue_gather(0)

        @pl.when(nch_w > 1)
        def _():
            issue_idx(1, 1)

        def step(i, b, nvalid):
            @pl.when(i + 1 < nvalid)
            def _():
                wait_idx(1 - b)
                @pl.when(i >= 1)
                def _():
                    wait_write(1 - b)
                issue_gather(1 - b)
            @pl.when(i < nvalid)
            def _():
                wait_gather(b)
                issue_write(i, b)
            @pl.when(i + 2 < nvalid)
            def _():
                issue_idx(i + 2, b)

        def pair(i2, _):
            step(2 * i2, 0, nch_w)
            step(2 * i2 + 1, 1, nch_w)
            return 0

        lax.fori_loop(0, (nch_w + 1) // 2, pair, 0)

        @pl.when(nch_w > 1)
        def _():
            wait_write(1)

        @pl.when(nch_w > 0)
        def _():
            wait_write(0)

    return k(db, eh, src2, dst2)


def _mm_body(x_ref, w_ref, b_ref, o_ref):
    o_ref[...] = jnp.dot(x_ref[...], w_ref[...],
                         preferred_element_type=jnp.float32) + b_ref[...]


def _mm(x, w, b, block):
    m, kdim = x.shape
    kout = w.shape[1]
    return pl.pallas_call(
        _mm_body,
        grid=(m // block,),
        in_specs=[pl.BlockSpec((block, kdim), lambda i: (i, 0)),
                  pl.BlockSpec((kdim, kout), lambda i: (0, 0)),
                  pl.BlockSpec((1, kout), lambda i: (0, 0))],
        out_specs=pl.BlockSpec((block, kout), lambda i: (i, 0)),
        out_shape=jax.ShapeDtypeStruct((m, kout), jnp.float32),
    )(x, w, b.reshape(1, kout))


def _edge_stage_body(gs_ref, gd_ref, ce_ref, enew_ref, con_ref):
    d = gs_ref[:, :_DIM]
    bb = gs_ref[:, _DIM:]
    e_ji = d + gd_ref[...] + ce_ref[...]
    sig = jax.nn.sigmoid(e_ji)
    enew_ref[...] = jax.nn.relu(e_ji)
    con_ref[:, :_DIM] = sig * bb
    con_ref[:, _DIM:] = sig


def _edge_stage_tc(gsrc, gdst, ce, block=2000):
    n = gsrc.shape[0]
    return pl.pallas_call(
        _edge_stage_body,
        grid=(n // block,),
        in_specs=[pl.BlockSpec((block, 2 * _DIM), lambda i: (i, 0)),
                  pl.BlockSpec((block, _DIM), lambda i: (i, 0)),
                  pl.BlockSpec((block, _DIM), lambda i: (i, 0))],
        out_specs=[pl.BlockSpec((block, _DIM), lambda i: (i, 0)),
                   pl.BlockSpec((block, 2 * _DIM), lambda i: (i, 0))],
        out_shape=[jax.ShapeDtypeStruct((n, _DIM), jnp.float32),
                   jax.ShapeDtypeStruct((n, 2 * _DIM), jnp.float32)],
    )(gsrc, gdst, ce)


def _node_stage_body(ah_ref, acc_ref, h_ref):
    num = acc_ref[:, :_DIM]
    den = acc_ref[:, _DIM:]
    h_ref[...] = jax.nn.relu(ah_ref[...] + num / (den + 1e-6))


def _node_stage(ah, acc, block=2000):
    n = ah.shape[0]
    return pl.pallas_call(
        _node_stage_body,
        grid=(n // block,),
        in_specs=[pl.BlockSpec((block, _DIM), lambda i: (i, 0)),
                  pl.BlockSpec((block, 2 * _DIM), lambda i: (i, 0))],
        out_specs=pl.BlockSpec((block, _DIM), lambda i: (i, 0)),
        out_shape=jax.ShapeDtypeStruct((n, _DIM), jnp.float32),
    )(ah, acc)


def _gated_layer(h, e, edge_index, p, n_nodes, n_edges, node_block,
                 rng, n_pass):
    src = edge_index[0]
    dst = edge_index[1]
    wdb = jnp.concatenate([p['D'][0], p['B'][0]], axis=1)
    bdb = jnp.concatenate([p['D'][1], p['B'][1]], axis=0)
    db = _mm(h, wdb, bdb, block=min(2000, n_nodes // 5))
    Eh = _mm(h, p['E'][0], p['E'][1], block=min(2000, n_nodes // 5))
    Ah = _mm(h, p['A'][0], p['A'][1], block=min(2000, n_nodes // 5))
    Ce = _mm(e, p['C'][0], p['C'][1], block=2000)
    src2 = src.reshape(n_edges // _C, _C)
    dst2 = dst.reshape(n_edges // _C, _C)
    gsrc, gdst = _gather_stage_sc(db, Eh, src2, dst2, n_edges)
    e_new, contrib = _edge_stage_tc(gsrc, gdst, Ce)
    acc = jax.ops.segment_sum(contrib, dst, num_segments=n_nodes)
    h_new = _node_stage(Ah, acc, block=node_block)
    return h_new, e_new


def kernel(node_feats, edge_feats, angle_feats, graph_edge_index, line_graph_edge_index, params):
    h, e = _gated_layer(node_feats, edge_feats, graph_edge_index,
                        params['node_update'], _N_NODES, _N_EDGES, node_block=2000,
                        rng=5120, n_pass=1)
    e, a = _gated_layer(e, angle_feats, line_graph_edge_index,
                        params['edge_update'], _N_EDGES, _N_ANGLES, node_block=2000,
                        rng=5120, n_pass=16)
    return (h, e, a)


# speedup vs baseline: 1.9494x; 1.0612x over previous
import functools

import jax
import jax.numpy as jnp
from jax import lax
from jax.experimental import pallas as pl
from jax.experimental.pallas import tpu as pltpu
from jax.experimental.pallas import tpu_sc as plsc

_N_NODES = 10000
_N_EDGES = 160000
_N_ANGLES = 320000
_DIM = 128

_NC = 2
_NS = 16
_NW = _NC * _NS
_C = 128

_mesh = plsc.VectorSubcoreMesh(
    core_axis_name="c", subcore_axis_name="s", num_cores=_NC, num_subcores=_NS)


def _gather_stage_sc(db, eh, src2, dst2, n_edges):
    nch = n_edges // _C

    @functools.partial(
        pl.kernel,
        out_type=(
            jax.ShapeDtypeStruct((n_edges, 2 * _DIM), jnp.float32),
            jax.ShapeDtypeStruct((n_edges, _DIM), jnp.float32),
        ),
        mesh=_mesh,
        scratch_types=[
            pltpu.VMEM((_C,), jnp.int32), pltpu.VMEM((_C,), jnp.int32),
            pltpu.VMEM((_C,), jnp.int32), pltpu.VMEM((_C,), jnp.int32),
            pltpu.VMEM((_C, 2 * _DIM), jnp.float32),
            pltpu.VMEM((_C, 2 * _DIM), jnp.float32),
            pltpu.VMEM((_C, _DIM), jnp.float32),
            pltpu.VMEM((_C, _DIM), jnp.float32),
            pltpu.SemaphoreType.DMA, pltpu.SemaphoreType.DMA,
            pltpu.SemaphoreType.DMA, pltpu.SemaphoreType.DMA,
            pltpu.SemaphoreType.DMA, pltpu.SemaphoreType.DMA,
        ],
    )
    def k(db_hbm, eh_hbm, src_hbm, dst_hbm, gsrc_hbm, gdst_hbm,
          src0, src1, dst0, dst1, gs0, gs1, gd0, gd1,
          si0, si1, sg0, sg1, sw0, sw1):
        wid = lax.axis_index("s") * _NC + lax.axis_index("c")
        nch_w = (nch - wid + _NW - 1) // _NW
        srcs, dsts = (src0, src1), (dst0, dst1)
        gss, gds = (gs0, gs1), (gd0, gd1)
        sis, sgs, sws = (si0, si1), (sg0, sg1), (sw0, sw1)

        def ch_of(i):
            return wid + i * _NW

        def issue_idx(i, b):
            ch = ch_of(i)
            pltpu.async_copy(src_hbm.at[ch], srcs[b], sis[b])
            pltpu.async_copy(dst_hbm.at[ch], dsts[b], sis[b])

        def wait_idx(b):
            pltpu.make_async_copy(src_hbm.at[0], srcs[b], sis[b]).wait()
            pltpu.make_async_copy(dst_hbm.at[0], dsts[b], sis[b]).wait()

        def issue_gather(b):
            pltpu.async_copy(db_hbm.at[srcs[b]], gss[b], sgs[b])
            pltpu.async_copy(eh_hbm.at[dsts[b]], gds[b], sgs[b])

        def wait_gather(b):
            pltpu.make_async_copy(db_hbm.at[srcs[b]], gss[b], sgs[b]).wait()
            pltpu.make_async_copy(eh_hbm.at[dsts[b]], gds[b], sgs[b]).wait()

        def issue_write(i, b):
            off = ch_of(i) * _C
            pltpu.async_copy(gss[b], gsrc_hbm.at[pl.ds(off, _C)], sws[b])
            pltpu.async_copy(gds[b], gdst_hbm.at[pl.ds(off, _C)], sws[b])

        def wait_write(b):
            pltpu.make_async_copy(gss[b], gsrc_hbm.at[pl.ds(0, _C)], sws[b]).wait()
            pltpu.make_async_copy(gds[b], gdst_hbm.at[pl.ds(0, _C)], sws[b]).wait()

        @pl.when(nch_w > 0)
        def _():
            issue_idx(0, 0)
            wait_idx(0)
            issue_gather(0)

        @pl.when(nch_w > 1)
        def _():
            issue_idx(1, 1)

        def step(i, b, nvalid):
            @pl.when(i + 1 < nvalid)
            def _():
                wait_idx(1 - b)
                @pl.when(i >= 1)
                def _():
                    wait_write(1 - b)
                issue_gather(1 - b)
            @pl.when(i < nvalid)
            def _():
                wait_gather(b)
                issue_write(i, b)
            @pl.when(i + 2 < nvalid)
            def _():
                issue_idx(i + 2, b)

        def pair(i2, _):
            step(2 * i2, 0, nch_w)
            step(2 * i2 + 1, 1, nch_w)
            return 0

        lax.fori_loop(0, (nch_w + 1) // 2, pair, 0)

        @pl.when(nch_w > 1)
        def _():
            wait_write(1)

        @pl.when(nch_w > 0)
        def _():
            wait_write(0)

    return k(db, eh, src2, dst2)


def _mm_body(x_ref, w_ref, b_ref, o_ref):
    o_ref[...] = jnp.dot(x_ref[...], w_ref[...],
                         preferred_element_type=jnp.float32) + b_ref[...]


def _mm(x, w, b, block):
    m, kdim = x.shape
    kout = w.shape[1]
    return pl.pallas_call(
        _mm_body,
        grid=(m // block,),
        in_specs=[pl.BlockSpec((block, kdim), lambda i: (i, 0)),
                  pl.BlockSpec((kdim, kout), lambda i: (0, 0)),
                  pl.BlockSpec((1, kout), lambda i: (0, 0))],
        out_specs=pl.BlockSpec((block, kout), lambda i: (i, 0)),
        out_shape=jax.ShapeDtypeStruct((m, kout), jnp.float32),
    )(x, w, b.reshape(1, kout))


def _edge_stage_body(gs_ref, gd_ref, e_ref, wc_ref, bc_ref, enew_ref, con_ref):
    ce = jnp.dot(e_ref[...], wc_ref[...],
                 preferred_element_type=jnp.float32) + bc_ref[...]
    d = gs_ref[:, :_DIM]
    bb = gs_ref[:, _DIM:]
    e_ji = d + gd_ref[...] + ce
    sig = jax.nn.sigmoid(e_ji)
    enew_ref[...] = jax.nn.relu(e_ji)
    con_ref[:, :_DIM] = sig * bb
    con_ref[:, _DIM:] = sig


def _edge_stage_tc(gsrc, gdst, efeat, wc, bc, block=2000):
    n = gsrc.shape[0]
    return pl.pallas_call(
        _edge_stage_body,
        grid=(n // block,),
        in_specs=[pl.BlockSpec((block, 2 * _DIM), lambda i: (i, 0)),
                  pl.BlockSpec((block, _DIM), lambda i: (i, 0)),
                  pl.BlockSpec((block, _DIM), lambda i: (i, 0)),
                  pl.BlockSpec((_DIM, _DIM), lambda i: (0, 0)),
                  pl.BlockSpec((1, _DIM), lambda i: (0, 0))],
        out_specs=[pl.BlockSpec((block, _DIM), lambda i: (i, 0)),
                   pl.BlockSpec((block, 2 * _DIM), lambda i: (i, 0))],
        out_shape=[jax.ShapeDtypeStruct((n, _DIM), jnp.float32),
                   jax.ShapeDtypeStruct((n, 2 * _DIM), jnp.float32)],
    )(gsrc, gdst, efeat, wc, bc.reshape(1, _DIM))


def _node_stage_body(ah_ref, acc_ref, h_ref):
    num = acc_ref[:, :_DIM]
    den = acc_ref[:, _DIM:]
    h_ref[...] = jax.nn.relu(ah_ref[...] + num / (den + 1e-6))


def _node_stage(ah, acc, block=2000):
    n = ah.shape[0]
    return pl.pallas_call(
        _node_stage_body,
        grid=(n // block,),
        in_specs=[pl.BlockSpec((block, _DIM), lambda i: (i, 0)),
                  pl.BlockSpec((block, 2 * _DIM), lambda i: (i, 0))],
        out_specs=pl.BlockSpec((block, _DIM), lambda i: (i, 0)),
        out_shape=jax.ShapeDtypeStruct((n, _DIM), jnp.float32),
    )(ah, acc)


def _gated_layer(h, e, edge_index, p, n_nodes, n_edges, node_block,
                 rng, n_pass):
    src = edge_index[0]
    dst = edge_index[1]
    wdb = jnp.concatenate([p['D'][0], p['B'][0]], axis=1)
    bdb = jnp.concatenate([p['D'][1], p['B'][1]], axis=0)
    db = _mm(h, wdb, bdb, block=min(2000, n_nodes // 5))
    Eh = _mm(h, p['E'][0], p['E'][1], block=min(2000, n_nodes // 5))
    Ah = _mm(h, p['A'][0], p['A'][1], block=min(2000, n_nodes // 5))
    src2 = src.reshape(n_edges // _C, _C)
    dst2 = dst.reshape(n_edges // _C, _C)
    gsrc, gdst = _gather_stage_sc(db, Eh, src2, dst2, n_edges)
    e_new, contrib = _edge_stage_tc(gsrc, gdst, e, p['C'][0], p['C'][1])
    acc = jax.ops.segment_sum(contrib, dst, num_segments=n_nodes)
    h_new = _node_stage(Ah, acc, block=node_block)
    return h_new, e_new


def kernel(node_feats, edge_feats, angle_feats, graph_edge_index, line_graph_edge_index, params):
    h, e = _gated_layer(node_feats, edge_feats, graph_edge_index,
                        params['node_update'], _N_NODES, _N_EDGES, node_block=2000,
                        rng=5120, n_pass=1)
    e, a = _gated_layer(e, angle_feats, line_graph_edge_index,
                        params['edge_update'], _N_EDGES, _N_ANGLES, node_block=2000,
                        rng=5120, n_pass=16)
    return (h, e, a)
